# baseline (speedup 1.0000x reference)
import jax
import jax.numpy as jnp
from jax.experimental import pallas as pl

_B, _S, _V, _E, _T = 4, 2048, 1000, 768, 2
_TM = 512


def _body(x_ref, tt_ref, w_ref, pos_ref, tyt_ref, gb_ref, o_ref):
    x = x_ref[...]
    w = w_ref[...]
    y = jnp.dot(x, w, preferred_element_type=jnp.float32)

    tt = tt_ref[0, 0, :].astype(jnp.float32)[:, None]
    ty0 = tyt_ref[0:1, :]
    ty1 = tyt_ref[1:2, :]
    y = y + pos_ref[...] + ty0 + tt * (ty1 - ty0)

    mean = jnp.mean(y, axis=1, keepdims=True)
    yc = y - mean
    var = jnp.mean(yc * yc, axis=1, keepdims=True)
    inv = jax.lax.rsqrt(var + 1e-3)
    o_ref[...] = yc * inv * gb_ref[0:1, :] + gb_ref[1:2, :]


def kernel(input_ids, token_type_ids, token_embedding, position_table, type_table, gamma, beta):
    B, S, V = input_ids.shape
    E = token_embedding.shape[1]
    M = B * S
    n_tiles = M // _TM
    s_tiles = S // _TM

    x = input_ids.reshape(M, V).astype(jnp.bfloat16)
    tt = token_type_ids.reshape(n_tiles, 1, _TM)
    token_embedding = token_embedding.astype(jnp.bfloat16)
    gb = jnp.stack([gamma, beta])

    out = pl.pallas_call(
        _body,
        grid=(n_tiles,),
        in_specs=[
            pl.BlockSpec((_TM, V), lambda i: (i, 0)),
            pl.BlockSpec((1, 1, _TM), lambda i: (i, 0, 0)),
            pl.BlockSpec((V, E), lambda i: (0, 0)),
            pl.BlockSpec((_TM, E), lambda i: (i % s_tiles, 0)),
            pl.BlockSpec((_T, E), lambda i: (0, 0)),
            pl.BlockSpec((2, E), lambda i: (0, 0)),
        ],
        out_specs=pl.BlockSpec((_TM, E), lambda i: (i, 0)),
        out_shape=jax.ShapeDtypeStruct((M, E), jnp.float32),
    )(x, tt, token_embedding, position_table, type_table, gb)

    return out.reshape(B, S, E)


# speedup vs baseline: 1.0168x; 1.0168x over previous
import jax
import jax.numpy as jnp
from jax.experimental import pallas as pl

_B, _S, _V, _E, _T = 4, 2048, 1000, 768, 2
_TM = 512


def _body(x_ref, tt_ref, w_ref, pos_ref, tyt_ref, gb_ref, o_ref):
    x = x_ref[...].astype(jnp.bfloat16)
    w = w_ref[...]
    y = jnp.dot(x, w, preferred_element_type=jnp.float32)

    tt = tt_ref[0, 0, :].astype(jnp.float32)[:, None]
    ty0 = tyt_ref[0:1, :]
    ty1 = tyt_ref[1:2, :]
    y = y + pos_ref[...] + ty0 + tt * (ty1 - ty0)

    mean = jnp.mean(y, axis=1, keepdims=True)
    yc = y - mean
    var = jnp.mean(yc * yc, axis=1, keepdims=True)
    inv = jax.lax.rsqrt(var + 1e-3)
    o_ref[...] = yc * inv * gb_ref[0:1, :] + gb_ref[1:2, :]


def kernel(input_ids, token_type_ids, token_embedding, position_table, type_table, gamma, beta):
    B, S, V = input_ids.shape
    E = token_embedding.shape[1]
    M = B * S
    n_tiles = M // _TM
    s_tiles = S // _TM

    x = input_ids.reshape(M, V)
    tt = token_type_ids.reshape(n_tiles, 1, _TM)
    token_embedding = token_embedding.astype(jnp.bfloat16)
    gb = jnp.stack([gamma, beta])

    out = pl.pallas_call(
        _body,
        grid=(n_tiles,),
        in_specs=[
            pl.BlockSpec((_TM, V), lambda i: (i, 0)),
            pl.BlockSpec((1, 1, _TM), lambda i: (i, 0, 0)),
            pl.BlockSpec((V, E), lambda i: (0, 0)),
            pl.BlockSpec((_TM, E), lambda i: (i % s_tiles, 0)),
            pl.BlockSpec((_T, E), lambda i: (0, 0)),
            pl.BlockSpec((2, E), lambda i: (0, 0)),
        ],
        out_specs=pl.BlockSpec((_TM, E), lambda i: (i, 0)),
        out_shape=jax.ShapeDtypeStruct((M, E), jnp.float32),
    )(x, tt, token_embedding, position_table, type_table, gb)

    return out.reshape(B, S, E)


# speedup vs baseline: 1.0363x; 1.0192x over previous
import jax
import jax.numpy as jnp
from jax.experimental import pallas as pl

_B, _S, _V, _E, _T = 4, 2048, 1000, 768, 2
_TM = 512


def _body(x_ref, tt_ref, w_ref, pos_ref, tyt_ref, gb_ref, o_ref, *, s_tiles):
    y = jnp.dot(x_ref[...], w_ref[...], preferred_element_type=jnp.float32)

    s_idx = pl.program_id(0) % s_tiles
    pos = pos_ref[pl.ds(s_idx * _TM, _TM), :]
    tt = tt_ref[0, 0, :].astype(jnp.float32)[:, None]
    ty0 = tyt_ref[0:1, :]
    ty1 = tyt_ref[1:2, :]
    y = y + pos + ty0 + tt * (ty1 - ty0)

    mean = jnp.mean(y, axis=1, keepdims=True)
    yc = y - mean
    var = jnp.mean(yc * yc, axis=1, keepdims=True)
    inv = jax.lax.rsqrt(var + 1e-3)
    o_ref[...] = yc * inv * gb_ref[0:1, :] + gb_ref[1:2, :]


def kernel(input_ids, token_type_ids, token_embedding, position_table, type_table, gamma, beta):
    B, S, V = input_ids.shape
    E = token_embedding.shape[1]
    M = B * S
    n_tiles = M // _TM
    s_tiles = S // _TM

    x = input_ids.reshape(M, V)
    tt = token_type_ids.reshape(n_tiles, 1, _TM)
    gb = jnp.stack([gamma, beta])

    import functools
    body = functools.partial(_body, s_tiles=s_tiles)

    out = pl.pallas_call(
        body,
        grid=(n_tiles,),
        in_specs=[
            pl.BlockSpec((_TM, V), lambda i: (i, 0)),
            pl.BlockSpec((1, 1, _TM), lambda i: (i, 0, 0)),
            pl.BlockSpec((V, E), lambda i: (0, 0)),
            pl.BlockSpec((S, E), lambda i: (0, 0)),
            pl.BlockSpec((_T, E), lambda i: (0, 0)),
            pl.BlockSpec((2, E), lambda i: (0, 0)),
        ],
        out_specs=pl.BlockSpec((_TM, E), lambda i: (i, 0)),
        out_shape=jax.ShapeDtypeStruct((M, E), jnp.float32),
    )(x, tt, token_embedding, position_table, type_table, gb)

    return out.reshape(B, S, E)


# speedup vs baseline: 1.1276x; 1.0881x over previous
import functools

import jax
import jax.numpy as jnp
from jax.experimental import pallas as pl

_B, _S, _V, _E, _T = 4, 2048, 1000, 768, 2
_TM = 512


def _body(x_ref, tt_ref, w_ref, pos_ref, tyt_ref, gb_ref, o_ref, *, s_tiles):
    y = jnp.dot(x_ref[0], w_ref[...], preferred_element_type=jnp.float32)

    s_idx = pl.program_id(0) % s_tiles
    pos = pos_ref[pl.ds(s_idx * _TM, _TM), :]
    tt = tt_ref[0, 0, :].astype(jnp.float32)[:, None]
    ty0 = tyt_ref[0:1, :]
    ty1 = tyt_ref[1:2, :]
    y = y + pos + ty0 + tt * (ty1 - ty0)

    mean = jnp.mean(y, axis=1, keepdims=True)
    yc = y - mean
    var = jnp.mean(yc * yc, axis=1, keepdims=True)
    inv = jax.lax.rsqrt(var + 1e-3)
    o_ref[0] = yc * inv * gb_ref[0:1, :] + gb_ref[1:2, :]


def kernel(input_ids, token_type_ids, token_embedding, position_table, type_table, gamma, beta):
    B, S, V = input_ids.shape
    E = token_embedding.shape[1]
    n_tiles = (B * S) // _TM
    s_tiles = S // _TM

    tt = token_type_ids.reshape(n_tiles, 1, _TM)
    gb = jnp.stack([gamma, beta])

    body = functools.partial(_body, s_tiles=s_tiles)

    out = pl.pallas_call(
        body,
        grid=(n_tiles,),
        in_specs=[
            pl.BlockSpec((1, _TM, V), lambda i, s=s_tiles: (i // s, i % s, 0)),
            pl.BlockSpec((1, 1, _TM), lambda i: (i, 0, 0)),
            pl.BlockSpec((V, E), lambda i: (0, 0)),
            pl.BlockSpec((S, E), lambda i: (0, 0)),
            pl.BlockSpec((_T, E), lambda i: (0, 0)),
            pl.BlockSpec((2, E), lambda i: (0, 0)),
        ],
        out_specs=pl.BlockSpec((1, _TM, E), lambda i, s=s_tiles: (i // s, i % s, 0)),
        out_shape=jax.ShapeDtypeStruct((B, S, E), jnp.float32),
    )(input_ids, tt, token_embedding, position_table, type_table, gb)

    return out


# speedup vs baseline: 1.2020x; 1.0660x over previous
import functools

import jax
import jax.numpy as jnp
from jax.experimental import pallas as pl

_B, _S, _V, _E, _T = 4, 2048, 1000, 768, 2
_TM = 1024


def _body(x_ref, tt_ref, w_ref, pos_ref, tyt_ref, gb_ref, o_ref, *, s_tiles):
    y = jnp.dot(x_ref[0], w_ref[...], preferred_element_type=jnp.float32)

    s_idx = pl.program_id(0) % s_tiles
    pos = pos_ref[pl.ds(s_idx * _TM, _TM), :]
    tt = tt_ref[0, 0, :].astype(jnp.float32)[:, None]
    ty0 = tyt_ref[0:1, :]
    ty1 = tyt_ref[1:2, :]
    y = y + pos + ty0 + tt * (ty1 - ty0)

    mean = jnp.mean(y, axis=1, keepdims=True)
    yc = y - mean
    var = jnp.mean(yc * yc, axis=1, keepdims=True)
    inv = jax.lax.rsqrt(var + 1e-3)
    o_ref[0] = yc * inv * gb_ref[0:1, :] + gb_ref[1:2, :]


def kernel(input_ids, token_type_ids, token_embedding, position_table, type_table, gamma, beta):
    B, S, V = input_ids.shape
    E = token_embedding.shape[1]
    n_tiles = (B * S) // _TM
    s_tiles = S // _TM

    tt = token_type_ids.reshape(n_tiles, 1, _TM)
    gb = jnp.stack([gamma, beta])

    body = functools.partial(_body, s_tiles=s_tiles)

    out = pl.pallas_call(
        body,
        grid=(n_tiles,),
        in_specs=[
            pl.BlockSpec((1, _TM, V), lambda i, s=s_tiles: (i // s, i % s, 0)),
            pl.BlockSpec((1, 1, _TM), lambda i: (i, 0, 0)),
            pl.BlockSpec((V, E), lambda i: (0, 0)),
            pl.BlockSpec((S, E), lambda i: (0, 0)),
            pl.BlockSpec((_T, E), lambda i: (0, 0)),
            pl.BlockSpec((2, E), lambda i: (0, 0)),
        ],
        out_specs=pl.BlockSpec((1, _TM, E), lambda i, s=s_tiles: (i // s, i % s, 0)),
        out_shape=jax.ShapeDtypeStruct((B, S, E), jnp.float32),
    )(input_ids, tt, token_embedding, position_table, type_table, gb)

    return out
